# baseline (speedup 1.0000x reference)
import functools

import jax
import jax.numpy as jnp
from jax import lax
from jax.experimental import pallas as pl
from jax.experimental.pallas import tpu as pltpu
from jax.experimental.pallas import tpu_sc as plsc

_N = 10000
_D = 128
_NP = 10112
_NW = 32
_LB = 64
_CB = 162
_EP = _NW * _CB * _LB
_RPT = _NP // 16
_CHUNKS = tuple((o * _LB, min(_LB, _RPT - o * _LB))
                for o in range((_RPT + _LB - 1) // _LB))
_BM = 632


def _mm2_body(x_ref, wl_ref, bl_ref, wr_ref, br_ref, xl_ref, xr_ref):
    xb = x_ref[...]
    xl_ref[...] = jnp.dot(xb, wl_ref[...],
                          preferred_element_type=jnp.float32) + bl_ref[...]
    xr_ref[...] = jnp.dot(xb, wr_ref[...],
                          preferred_element_type=jnp.float32) + br_ref[...]


def _tc_transform(x, Wl, bl, Wr, br):
    bm = _BM
    grid = (_NP // bm,)
    return pl.pallas_call(
        _mm2_body,
        grid=grid,
        in_specs=[
            pl.BlockSpec((bm, _D), lambda i: (i, 0)),
            pl.BlockSpec((_D, _D), lambda i: (0, 0)),
            pl.BlockSpec((1, _D), lambda i: (0, 0)),
            pl.BlockSpec((_D, _D), lambda i: (0, 0)),
            pl.BlockSpec((1, _D), lambda i: (0, 0)),
        ],
        out_specs=[
            pl.BlockSpec((bm, _D), lambda i: (i, 0)),
            pl.BlockSpec((bm, _D), lambda i: (i, 0)),
        ],
        out_shape=[
            jax.ShapeDtypeStruct((_NP, _D), jnp.float32),
            jax.ShapeDtypeStruct((_NP, _D), jnp.float32),
        ],
    )(x, Wl, bl.reshape(1, _D), Wr, br.reshape(1, _D))


def _combine_transform_body(raw_ref, den_ref, b0_ref, wl_ref, bl_ref,
                            wr_ref, br_ref, xl_ref, xr_ref):
    r = raw_ref[0] + raw_ref[1]
    d = den_ref[0] + den_ref[1]
    h = jnp.maximum(r / (d + 1e-16) + b0_ref[...], 0.0)
    xl_ref[...] = jnp.dot(h, wl_ref[...],
                          preferred_element_type=jnp.float32) + bl_ref[...]
    xr_ref[...] = jnp.dot(h, wr_ref[...],
                          preferred_element_type=jnp.float32) + br_ref[...]


def _tc_combine_transform(raw, den, b0, Wl, bl, Wr, br):
    bm = _BM
    grid = (_NP // bm,)
    return pl.pallas_call(
        _combine_transform_body,
        grid=grid,
        in_specs=[
            pl.BlockSpec((2, bm, _D), lambda i: (0, i, 0)),
            pl.BlockSpec((2, bm, 1), lambda i: (0, i, 0)),
            pl.BlockSpec((1, _D), lambda i: (0, 0)),
            pl.BlockSpec((_D, _D), lambda i: (0, 0)),
            pl.BlockSpec((1, _D), lambda i: (0, 0)),
            pl.BlockSpec((_D, _D), lambda i: (0, 0)),
            pl.BlockSpec((1, _D), lambda i: (0, 0)),
        ],
        out_specs=[
            pl.BlockSpec((bm, _D), lambda i: (i, 0)),
            pl.BlockSpec((bm, _D), lambda i: (i, 0)),
        ],
        out_shape=[
            jax.ShapeDtypeStruct((_NP, _D), jnp.float32),
            jax.ShapeDtypeStruct((_NP, _D), jnp.float32),
        ],
    )(raw, den.reshape(2, _NP, 1), b0.reshape(1, _D), Wl,
      bl.reshape(1, _D), Wr, br.reshape(1, _D))


def _final_body(raw_ref, den_ref, b_ref, out_ref):
    r = raw_ref[0] + raw_ref[1]
    d = den_ref[0] + den_ref[1]
    out_ref[...] = r / (d + 1e-16) + b_ref[...]


def _tc_final(raw, den, b):
    bm = _BM
    grid = (_NP // bm,)
    return pl.pallas_call(
        _final_body,
        grid=grid,
        in_specs=[
            pl.BlockSpec((2, bm, _D), lambda i: (0, i, 0)),
            pl.BlockSpec((2, bm, 1), lambda i: (0, i, 0)),
            pl.BlockSpec((1, _D), lambda i: (0, 0)),
        ],
        out_specs=pl.BlockSpec((bm, _D), lambda i: (i, 0)),
        out_shape=jax.ShapeDtypeStruct((_NP, _D), jnp.float32),
    )(raw, den.reshape(2, _NP, 1), b.reshape(1, _D))


def _sc_body(xl_hbm, xr_hbm, src_hbm, dst_hbm, att_hbm,
             raw_hbm, den0_hbm, den1_hbm,
             srcb0, srcb1, dstb0, dstb1, xl0, xl1, xr0, xr1, ab0, ab1, attb,
             spout, spden,
             sis0, sis1, sid0, sid1, sgx0, sgx1, sgr0, sgr1, ssc0, ssc1):
    c = lax.axis_index("c")
    s = lax.axis_index("s")
    w = c * 16 + s
    srcb = (srcb0, srcb1)
    dstb = (dstb0, dstb1)
    xlb = (xl0, xl1)
    xrb = (xr0, xr1)
    ab = (ab0, ab1)
    sis = (sis0, sis1)
    sid = (sid0, sid1)
    sgx = (sgx0, sgx1)
    sgr = (sgr0, sgr1)
    ssc = (ssc0, ssc1)

    pltpu.sync_copy(att_hbm, attb)

    zero16 = jnp.zeros((16,), jnp.float32)
    lane = lax.iota(jnp.int32, 16)

    def _zrow(r, carry):
        for j in range(8):
            xl0[r, pl.ds(j * 16, 16)] = zero16
        return carry
    lax.fori_loop(0, _LB, _zrow, 0)

    base = s * _RPT
    for off, sz in _CHUNKS:
        pltpu.sync_copy(xl0.at[pl.ds(0, sz)], spout.at[pl.ds(base + off, sz)])

    @pl.when(s == 0)
    def _():
        for i in range(_NP // _D):
            pltpu.sync_copy(xl0.at[0], spden.at[pl.ds(i * _D, _D)])
    plsc.subcore_barrier()

    def _compute(p):
        xlrows = xlb[p]
        xrrows = xrb[p]
        abuf = ab[p]

        def _group(g, gcarry):
            rowids = g * 16 + lane

            def _feat(j, accs):
                res = list(accs)
                for k in range(16):
                    ci = j * 16 + k
                    cvec = (ci + lane) & (_D - 1)
                    xlv = plsc.load_gather(xlrows, [rowids, cvec])
                    xrv = plsc.load_gather(xrrows, [rowids, cvec])
                    t = xlv + xrv
                    lk = jnp.maximum(t, 0.2 * t)
                    attc = plsc.load_gather(attb, [cvec])
                    res[k % 4] = res[k % 4] + lk * attc
                return tuple(res)
            acc = lax.fori_loop(0, 8, _feat, (zero16, zero16, zero16, zero16))
            av = jnp.exp((acc[0] + acc[1]) + (acc[2] + acc[3]))
            abuf[pl.ds(g * 16, 16)] = av

            for k in range(16):
                e = g * 16 + k
                a = av[k]
                for jj in range(8):
                    sl = pl.ds(jj * 16, 16)
                    xlrows[e, sl] = xlrows[e, sl] * a
            return gcarry
        lax.fori_loop(0, _LB // 16, _group, 0)

    pltpu.sync_copy(src_hbm.at[w, 0], srcb0)
    pltpu.sync_copy(dst_hbm.at[w, 0], dstb0)
    pltpu.async_copy(xl_hbm.at[srcb0], xl0, sgx0)
    pltpu.async_copy(xr_hbm.at[dstb0], xr0, sgr0)

    def _pair(t2, carry):
        for p in (0, 1):
            q = 1 - p
            t = 2 * t2 + p
            def _prefetch_idx():
                pltpu.async_copy(src_hbm.at[w, t + 1], srcb[q], sis[q])
                pltpu.async_copy(dst_hbm.at[w, t + 1], dstb[q], sid[q])

            def _fire_gathers():
                pltpu.make_async_copy(src_hbm.at[w, t + 1], srcb[q],
                                      sis[q]).wait()
                pltpu.make_async_copy(dst_hbm.at[w, t + 1], dstb[q],
                                      sid[q]).wait()
                pltpu.async_copy(xl_hbm.at[srcb[q]], xlb[q], sgx[q])
                pltpu.async_copy(xr_hbm.at[dstb[q]], xrb[q], sgr[q])
            if p == 0:
                _prefetch_idx()
            else:
                @pl.when(t2 < (_CB // 2) - 1)
                def _():
                    _prefetch_idx()

            pltpu.make_async_copy(xl_hbm.at[srcb[p]], xlb[p], sgx[p]).wait()
            pltpu.make_async_copy(xr_hbm.at[dstb[p]], xrb[p], sgr[p]).wait()
            _compute(p)
            d1 = pltpu.async_copy(ab[p], spden.at[dstb[p]], ssc[p],
                                  add=True)
            d2 = pltpu.async_copy(xlb[p], spout.at[dstb[p]], ssc[p],
                                  add=True)
            if p == 0:
                _fire_gathers()
            else:
                @pl.when(t2 < (_CB // 2) - 1)
                def _():
                    _fire_gathers()
            d1.wait()
            d2.wait()
        return carry
    lax.fori_loop(0, _CB // 2, _pair, 0)

    plsc.subcore_barrier()

    @pl.when((c == 0) & (s == 0))
    def _():
        pltpu.sync_copy(spden, den0_hbm)

    @pl.when((c == 1) & (s == 0))
    def _():
        pltpu.sync_copy(spden, den1_hbm)

    for off, sz in _CHUNKS:
        r0 = base + off
        pltpu.sync_copy(spout.at[pl.ds(r0, sz)],
                        raw_hbm.at[c, pl.ds(r0, sz)])


def _sc_edge_pass(xl, xr, src3, dst3, att):
    attb = att
    mesh = plsc.VectorSubcoreMesh(core_axis_name="c", subcore_axis_name="s")
    kern = pl.kernel(
        _sc_body,
        mesh=mesh,
        compiler_params=pltpu.CompilerParams(needs_layout_passes=False),
        out_type=[
            jax.ShapeDtypeStruct((2, _NP, _D), jnp.float32),
            jax.ShapeDtypeStruct((_NP,), jnp.float32),
            jax.ShapeDtypeStruct((_NP,), jnp.float32),
        ],
        scratch_types=(
            [pltpu.VMEM((_LB,), jnp.int32)] * 4
            + [pltpu.VMEM((_LB, _D), jnp.float32)] * 4
            + [pltpu.VMEM((_LB,), jnp.float32)] * 2
            + [pltpu.VMEM((_D,), jnp.float32)]
            + [pltpu.VMEM_SHARED((_NP, _D), jnp.float32),
               pltpu.VMEM_SHARED((_NP,), jnp.float32)]
            + [pltpu.SemaphoreType.DMA] * 10
        ),
    )
    raw, den0, den1 = kern(xl, xr, src3, dst3, attb)
    return raw, jnp.stack([den0, den1])


def kernel(x, edge_index, Wl1, bl1, Wr1, br1, att1, bias1,
           Wl2, bl2, Wr2, br2, att2, bias2):
    n = _N
    loop = jnp.arange(n, dtype=edge_index.dtype)
    pad = _EP - (edge_index.shape[1] + n)
    padv = (n + jnp.arange(pad, dtype=edge_index.dtype) % (_NP - n))
    src = jnp.concatenate([edge_index[0], loop, padv])
    dst = jnp.concatenate([edge_index[1], loop, padv])
    src3 = src.reshape(_NW, _CB, _LB)
    dst3 = dst.reshape(_NW, _CB, _LB)

    x_pad = jnp.pad(x, ((0, _NP - n), (0, 0)))

    xl1, xr1 = _tc_transform(x_pad, Wl1, bl1, Wr1, br1)
    raw1, den1 = _sc_edge_pass(xl1, xr1, src3, dst3, att1)
    xl2, xr2 = _tc_combine_transform(raw1, den1, bias1, Wl2, bl2, Wr2, br2)
    raw2, den2 = _sc_edge_pass(xl2, xr2, src3, dst3, att2)
    out = _tc_final(raw2, den2, bias2)
    return out[:n]


# speedup vs baseline: 1.0819x; 1.0819x over previous
import functools

import jax
import jax.numpy as jnp
from jax import lax
from jax.experimental import pallas as pl
from jax.experimental.pallas import tpu as pltpu
from jax.experimental.pallas import tpu_sc as plsc

_N = 10000
_D = 128
_NP = 10112
_NW = 32
_LB = 64
_CB = 162
_EP = _NW * _CB * _LB
_RPT = _NP // 16
_CHUNKS = tuple((o * _LB, min(_LB, _RPT - o * _LB))
                for o in range((_RPT + _LB - 1) // _LB))
_BM = 632


def _mm2_body(x_ref, wl_ref, bl_ref, wr_ref, br_ref, xl_ref, xr_ref):
    xb = x_ref[...]
    xl_ref[...] = jnp.dot(xb, wl_ref[...],
                          preferred_element_type=jnp.float32) + bl_ref[...]
    xr_ref[...] = jnp.dot(xb, wr_ref[...],
                          preferred_element_type=jnp.float32) + br_ref[...]


def _tc_transform(x, Wl, bl, Wr, br):
    bm = _BM
    grid = (_NP // bm,)
    return pl.pallas_call(
        _mm2_body,
        grid=grid,
        in_specs=[
            pl.BlockSpec((bm, _D), lambda i: (i, 0)),
            pl.BlockSpec((_D, _D), lambda i: (0, 0)),
            pl.BlockSpec((1, _D), lambda i: (0, 0)),
            pl.BlockSpec((_D, _D), lambda i: (0, 0)),
            pl.BlockSpec((1, _D), lambda i: (0, 0)),
        ],
        out_specs=[
            pl.BlockSpec((bm, _D), lambda i: (i, 0)),
            pl.BlockSpec((bm, _D), lambda i: (i, 0)),
        ],
        out_shape=[
            jax.ShapeDtypeStruct((_NP, _D), jnp.float32),
            jax.ShapeDtypeStruct((_NP, _D), jnp.float32),
        ],
    )(x, Wl, bl.reshape(1, _D), Wr, br.reshape(1, _D))


def _combine_transform_body(raw_ref, den_ref, b0_ref, wl_ref, bl_ref,
                            wr_ref, br_ref, xl_ref, xr_ref):
    r = raw_ref[0] + raw_ref[1]
    d = den_ref[0] + den_ref[1]
    h = jnp.maximum(r / (d + 1e-16) + b0_ref[...], 0.0)
    xl_ref[...] = jnp.dot(h, wl_ref[...],
                          preferred_element_type=jnp.float32) + bl_ref[...]
    xr_ref[...] = jnp.dot(h, wr_ref[...],
                          preferred_element_type=jnp.float32) + br_ref[...]


def _tc_combine_transform(raw, den, b0, Wl, bl, Wr, br):
    bm = _BM
    grid = (_NP // bm,)
    return pl.pallas_call(
        _combine_transform_body,
        grid=grid,
        in_specs=[
            pl.BlockSpec((2, bm, _D), lambda i: (0, i, 0)),
            pl.BlockSpec((2, bm, 1), lambda i: (0, i, 0)),
            pl.BlockSpec((1, _D), lambda i: (0, 0)),
            pl.BlockSpec((_D, _D), lambda i: (0, 0)),
            pl.BlockSpec((1, _D), lambda i: (0, 0)),
            pl.BlockSpec((_D, _D), lambda i: (0, 0)),
            pl.BlockSpec((1, _D), lambda i: (0, 0)),
        ],
        out_specs=[
            pl.BlockSpec((bm, _D), lambda i: (i, 0)),
            pl.BlockSpec((bm, _D), lambda i: (i, 0)),
        ],
        out_shape=[
            jax.ShapeDtypeStruct((_NP, _D), jnp.float32),
            jax.ShapeDtypeStruct((_NP, _D), jnp.float32),
        ],
    )(raw, den.reshape(2, _NP, 1), b0.reshape(1, _D), Wl,
      bl.reshape(1, _D), Wr, br.reshape(1, _D))


def _final_body(raw_ref, den_ref, b_ref, out_ref):
    r = raw_ref[0] + raw_ref[1]
    d = den_ref[0] + den_ref[1]
    out_ref[...] = r / (d + 1e-16) + b_ref[...]


def _tc_final(raw, den, b):
    bm = _BM
    grid = (_NP // bm,)
    return pl.pallas_call(
        _final_body,
        grid=grid,
        in_specs=[
            pl.BlockSpec((2, bm, _D), lambda i: (0, i, 0)),
            pl.BlockSpec((2, bm, 1), lambda i: (0, i, 0)),
            pl.BlockSpec((1, _D), lambda i: (0, 0)),
        ],
        out_specs=pl.BlockSpec((bm, _D), lambda i: (i, 0)),
        out_shape=jax.ShapeDtypeStruct((_NP, _D), jnp.float32),
    )(raw, den.reshape(2, _NP, 1), b.reshape(1, _D))


def _sc_body(xl_hbm, xr_hbm, src_hbm, dst_hbm, att_hbm,
             raw_hbm, den0_hbm, den1_hbm,
             srcb0, srcb1, dstb0, dstb1, xl0, xl1, xr0, xr1, ab0, ab1, attb,
             spout, spden,
             sis0, sis1, sid0, sid1, sgx0, sgx1, sgr0, sgr1, ssc0, ssc1):
    c = lax.axis_index("c")
    s = lax.axis_index("s")
    w = c * 16 + s
    srcb = (srcb0, srcb1)
    dstb = (dstb0, dstb1)
    xlb = (xl0, xl1)
    xrb = (xr0, xr1)
    ab = (ab0, ab1)
    sis = (sis0, sis1)
    sid = (sid0, sid1)
    sgx = (sgx0, sgx1)
    sgr = (sgr0, sgr1)
    ssc = (ssc0, ssc1)

    pltpu.sync_copy(att_hbm, attb)

    zero16 = jnp.zeros((16,), jnp.float32)
    lane = lax.iota(jnp.int32, 16)

    def _zrow(r, carry):
        for j in range(8):
            xl0[r, pl.ds(j * 16, 16)] = zero16
        return carry
    lax.fori_loop(0, _LB, _zrow, 0)

    base = s * _RPT
    for off, sz in _CHUNKS:
        pltpu.sync_copy(xl0.at[pl.ds(0, sz)], spout.at[pl.ds(base + off, sz)])

    @pl.when(s == 0)
    def _():
        for i in range(_NP // _D):
            pltpu.sync_copy(xl0.at[0], spden.at[pl.ds(i * _D, _D)])
    plsc.subcore_barrier()

    def _compute(p):
        xlrows = xlb[p]
        xrrows = xrb[p]
        abuf = ab[p]

        def _group(g, gcarry):
            rowids = g * 16 + lane

            def _feat(j, accs):
                res = list(accs)
                for k in range(16):
                    ci = j * 16 + k
                    cvec = (ci + lane) & (_D - 1)
                    xlv = plsc.load_gather(xlrows, [rowids, cvec])
                    xrv = plsc.load_gather(xrrows, [rowids, cvec])
                    t = xlv + xrv
                    lk = jnp.maximum(t, 0.2 * t)
                    attc = plsc.load_gather(attb, [cvec])
                    res[k % 4] = res[k % 4] + lk * attc
                return tuple(res)
            acc = lax.fori_loop(0, 8, _feat, (zero16, zero16, zero16, zero16))
            av = jnp.exp((acc[0] + acc[1]) + (acc[2] + acc[3]))
            abuf[pl.ds(g * 16, 16)] = av

            for k in range(16):
                e = g * 16 + k
                a = av[k]
                for jj in range(8):
                    sl = pl.ds(jj * 16, 16)
                    xlrows[e, sl] = xlrows[e, sl] * a
            return gcarry
        lax.fori_loop(0, _LB // 16, _group, 0)

    pltpu.sync_copy(src_hbm.at[w, 0], srcb0)
    pltpu.sync_copy(dst_hbm.at[w, 0], dstb0)
    pltpu.async_copy(xl_hbm.at[srcb0], xl0, sgx0)
    pltpu.async_copy(xr_hbm.at[dstb0], xr0, sgr0)

    def _pair(t2, carry):
        for p in (0, 1):
            q = 1 - p
            t = 2 * t2 + p
            def _prefetch_idx():
                pltpu.async_copy(src_hbm.at[w, t + 1], srcb[q], sis[q])
                pltpu.async_copy(dst_hbm.at[w, t + 1], dstb[q], sid[q])

            def _fire_gathers():
                pltpu.make_async_copy(src_hbm.at[w, t + 1], srcb[q],
                                      sis[q]).wait()
                pltpu.make_async_copy(dst_hbm.at[w, t + 1], dstb[q],
                                      sid[q]).wait()
                pltpu.async_copy(xl_hbm.at[srcb[q]], xlb[q], sgx[q])
                pltpu.async_copy(xr_hbm.at[dstb[q]], xrb[q], sgr[q])
            if p == 0:
                _prefetch_idx()
            else:
                @pl.when(t2 < (_CB // 2) - 1)
                def _():
                    _prefetch_idx()

            if p == 0:
                _fire_gathers()
            else:
                @pl.when(t2 < (_CB // 2) - 1)
                def _():
                    _fire_gathers()
            pltpu.make_async_copy(xl_hbm.at[srcb[p]], xlb[p], sgx[p]).wait()
            pltpu.make_async_copy(xr_hbm.at[dstb[p]], xrb[p], sgr[p]).wait()
            _compute(p)
            pltpu.sync_copy(ab[p], spden.at[dstb[p]], add=True)
            pltpu.sync_copy(xlb[p], spout.at[dstb[p]], add=True)
        return carry
    lax.fori_loop(0, _CB // 2, _pair, 0)

    plsc.subcore_barrier()

    @pl.when((c == 0) & (s == 0))
    def _():
        pltpu.sync_copy(spden, den0_hbm)

    @pl.when((c == 1) & (s == 0))
    def _():
        pltpu.sync_copy(spden, den1_hbm)

    for off, sz in _CHUNKS:
        r0 = base + off
        pltpu.sync_copy(spout.at[pl.ds(r0, sz)],
                        raw_hbm.at[c, pl.ds(r0, sz)])


def _sc_edge_pass(xl, xr, src3, dst3, att):
    attb = att
    mesh = plsc.VectorSubcoreMesh(core_axis_name="c", subcore_axis_name="s")
    kern = pl.kernel(
        _sc_body,
        mesh=mesh,
        compiler_params=pltpu.CompilerParams(needs_layout_passes=False),
        out_type=[
            jax.ShapeDtypeStruct((2, _NP, _D), jnp.float32),
            jax.ShapeDtypeStruct((_NP,), jnp.float32),
            jax.ShapeDtypeStruct((_NP,), jnp.float32),
        ],
        scratch_types=(
            [pltpu.VMEM((_LB,), jnp.int32)] * 4
            + [pltpu.VMEM((_LB, _D), jnp.float32)] * 4
            + [pltpu.VMEM((_LB,), jnp.float32)] * 2
            + [pltpu.VMEM((_D,), jnp.float32)]
            + [pltpu.VMEM_SHARED((_NP, _D), jnp.float32),
               pltpu.VMEM_SHARED((_NP,), jnp.float32)]
            + [pltpu.SemaphoreType.DMA] * 10
        ),
    )
    raw, den0, den1 = kern(xl, xr, src3, dst3, attb)
    return raw, jnp.stack([den0, den1])


def kernel(x, edge_index, Wl1, bl1, Wr1, br1, att1, bias1,
           Wl2, bl2, Wr2, br2, att2, bias2):
    n = _N
    loop = jnp.arange(n, dtype=edge_index.dtype)
    pad = _EP - (edge_index.shape[1] + n)
    padv = (n + jnp.arange(pad, dtype=edge_index.dtype) % (_NP - n))
    src = jnp.concatenate([edge_index[0], loop, padv])
    dst = jnp.concatenate([edge_index[1], loop, padv])
    src3 = src.reshape(_NW, _CB, _LB)
    dst3 = dst.reshape(_NW, _CB, _LB)

    x_pad = jnp.pad(x, ((0, _NP - n), (0, 0)))

    xl1, xr1 = _tc_transform(x_pad, Wl1, bl1, Wr1, br1)
    raw1, den1 = _sc_edge_pass(xl1, xr1, src3, dst3, att1)
    xl2, xr2 = _tc_combine_transform(raw1, den1, bias1, Wl2, bl2, Wr2, br2)
    raw2, den2 = _sc_edge_pass(xl2, xr2, src3, dst3, att2)
    out = _tc_final(raw2, den2, bias2)
    return out[:n]


# speedup vs baseline: 1.0847x; 1.0026x over previous
import functools

import jax
import jax.numpy as jnp
from jax import lax
from jax.experimental import pallas as pl
from jax.experimental.pallas import tpu as pltpu
from jax.experimental.pallas import tpu_sc as plsc

_N = 10000
_D = 128
_NP = 10112
_NW = 32
_LB = 64
_CB = 162
_EP = _NW * _CB * _LB
_RPT = _NP // 16
_CHUNKS = tuple((o * _LB, min(_LB, _RPT - o * _LB))
                for o in range((_RPT + _LB - 1) // _LB))
_BM = 632


def _mm2_body(x_ref, wl_ref, bl_ref, wr_ref, br_ref, xl_ref, xr_ref):
    xb = x_ref[...]
    xl_ref[...] = jnp.dot(xb, wl_ref[...],
                          preferred_element_type=jnp.float32) + bl_ref[...]
    xr_ref[...] = jnp.dot(xb, wr_ref[...],
                          preferred_element_type=jnp.float32) + br_ref[...]


def _tc_transform(x, Wl, bl, Wr, br):
    bm = _BM
    grid = (_NP // bm,)
    return pl.pallas_call(
        _mm2_body,
        grid=grid,
        in_specs=[
            pl.BlockSpec((bm, _D), lambda i: (i, 0)),
            pl.BlockSpec((_D, _D), lambda i: (0, 0)),
            pl.BlockSpec((1, _D), lambda i: (0, 0)),
            pl.BlockSpec((_D, _D), lambda i: (0, 0)),
            pl.BlockSpec((1, _D), lambda i: (0, 0)),
        ],
        out_specs=[
            pl.BlockSpec((bm, _D), lambda i: (i, 0)),
            pl.BlockSpec((bm, _D), lambda i: (i, 0)),
        ],
        out_shape=[
            jax.ShapeDtypeStruct((_NP, _D), jnp.float32),
            jax.ShapeDtypeStruct((_NP, _D), jnp.float32),
        ],
    )(x, Wl, bl.reshape(1, _D), Wr, br.reshape(1, _D))


def _combine_transform_body(raw_ref, den_ref, b0_ref, wl_ref, bl_ref,
                            wr_ref, br_ref, xl_ref, xr_ref):
    r = raw_ref[0] + raw_ref[1]
    d = den_ref[0] + den_ref[1]
    h = jnp.maximum(r / (d + 1e-16) + b0_ref[...], 0.0)
    xl_ref[...] = jnp.dot(h, wl_ref[...],
                          preferred_element_type=jnp.float32) + bl_ref[...]
    xr_ref[...] = jnp.dot(h, wr_ref[...],
                          preferred_element_type=jnp.float32) + br_ref[...]


def _tc_combine_transform(raw, den, b0, Wl, bl, Wr, br):
    bm = _BM
    grid = (_NP // bm,)
    return pl.pallas_call(
        _combine_transform_body,
        grid=grid,
        in_specs=[
            pl.BlockSpec((2, bm, _D), lambda i: (0, i, 0)),
            pl.BlockSpec((2, bm, 1), lambda i: (0, i, 0)),
            pl.BlockSpec((1, _D), lambda i: (0, 0)),
            pl.BlockSpec((_D, _D), lambda i: (0, 0)),
            pl.BlockSpec((1, _D), lambda i: (0, 0)),
            pl.BlockSpec((_D, _D), lambda i: (0, 0)),
            pl.BlockSpec((1, _D), lambda i: (0, 0)),
        ],
        out_specs=[
            pl.BlockSpec((bm, _D), lambda i: (i, 0)),
            pl.BlockSpec((bm, _D), lambda i: (i, 0)),
        ],
        out_shape=[
            jax.ShapeDtypeStruct((_NP, _D), jnp.float32),
            jax.ShapeDtypeStruct((_NP, _D), jnp.float32),
        ],
    )(raw, den.reshape(2, _NP, 1), b0.reshape(1, _D), Wl,
      bl.reshape(1, _D), Wr, br.reshape(1, _D))


def _final_body(raw_ref, den_ref, b_ref, out_ref):
    r = raw_ref[0] + raw_ref[1]
    d = den_ref[0] + den_ref[1]
    out_ref[...] = r / (d + 1e-16) + b_ref[...]


def _tc_final(raw, den, b):
    bm = _BM
    grid = (_NP // bm,)
    return pl.pallas_call(
        _final_body,
        grid=grid,
        in_specs=[
            pl.BlockSpec((2, bm, _D), lambda i: (0, i, 0)),
            pl.BlockSpec((2, bm, 1), lambda i: (0, i, 0)),
            pl.BlockSpec((1, _D), lambda i: (0, 0)),
        ],
        out_specs=pl.BlockSpec((bm, _D), lambda i: (i, 0)),
        out_shape=jax.ShapeDtypeStruct((_NP, _D), jnp.float32),
    )(raw, den.reshape(2, _NP, 1), b.reshape(1, _D))


def _sc_body(xl_hbm, xr_hbm, src_hbm, dst_hbm, att_hbm,
             raw_hbm, den0_hbm, den1_hbm,
             srcb0, srcb1, dstb0, dstb1, xl0, xl1, xr0, xr1, ab0, ab1, attb,
             spout, spden,
             sis0, sis1, sid0, sid1, sgx0, sgx1, sgr0, sgr1, ssc0, ssc1):
    c = lax.axis_index("c")
    s = lax.axis_index("s")
    w = c * 16 + s
    srcb = (srcb0, srcb1)
    dstb = (dstb0, dstb1)
    xlb = (xl0, xl1)
    xrb = (xr0, xr1)
    ab = (ab0, ab1)
    sis = (sis0, sis1)
    sid = (sid0, sid1)
    sgx = (sgx0, sgx1)
    sgr = (sgr0, sgr1)
    ssc = (ssc0, ssc1)

    pltpu.sync_copy(att_hbm, attb)

    zero16 = jnp.zeros((16,), jnp.float32)
    lane = lax.iota(jnp.int32, 16)

    def _zrow(r, carry):
        for j in range(8):
            xl0[r, pl.ds(j * 16, 16)] = zero16
        return carry
    lax.fori_loop(0, _LB, _zrow, 0)

    base = s * _RPT
    for off, sz in _CHUNKS:
        pltpu.sync_copy(xl0.at[pl.ds(0, sz)], spout.at[pl.ds(base + off, sz)])

    @pl.when(s == 0)
    def _():
        for i in range(_NP // _D):
            pltpu.sync_copy(xl0.at[0], spden.at[pl.ds(i * _D, _D)])
    plsc.subcore_barrier()

    def _compute(p):
        xlrows = xlb[p]
        xrrows = xrb[p]
        abuf = ab[p]

        def _group(g, gcarry):
            rowids = g * 16 + lane

            def _feat(j, accs):
                res = list(accs)
                for k in range(16):
                    ci = j * 16 + k
                    cvec = (ci + lane) & (_D - 1)
                    xlv = plsc.load_gather(xlrows, [rowids, cvec])
                    xrv = plsc.load_gather(xrrows, [rowids, cvec])
                    t = xlv + xrv
                    lk = jnp.maximum(t, 0.2 * t)
                    attc = plsc.load_gather(attb, [cvec])
                    res[k % 4] = res[k % 4] + lk * attc
                return tuple(res)
            acc = lax.fori_loop(0, 8, _feat, (zero16, zero16, zero16, zero16))
            av = jnp.exp((acc[0] + acc[1]) + (acc[2] + acc[3]))
            abuf[pl.ds(g * 16, 16)] = av

            for k in range(16):
                e = g * 16 + k
                a = av[k]
                for jj in range(8):
                    sl = pl.ds(jj * 16, 16)
                    xlrows[e, sl] = xlrows[e, sl] * a
            return gcarry
        lax.fori_loop(0, _LB // 16, _group, 0)

    pltpu.sync_copy(src_hbm.at[w, 0], srcb0)
    pltpu.sync_copy(dst_hbm.at[w, 0], dstb0)
    pltpu.async_copy(xl_hbm.at[srcb0], xl0, sgx0)
    pltpu.async_copy(xr_hbm.at[dstb0], xr0, sgr0)

    def _pair(t2, carry):
        for p in (0, 1):
            q = 1 - p
            t = 2 * t2 + p
            def _prefetch_idx():
                pltpu.async_copy(src_hbm.at[w, t + 1], srcb[q], sis[q])
                pltpu.async_copy(dst_hbm.at[w, t + 1], dstb[q], sid[q])

            def _fire_gathers():
                pltpu.make_async_copy(src_hbm.at[w, t + 1], srcb[q],
                                      sis[q]).wait()
                pltpu.make_async_copy(dst_hbm.at[w, t + 1], dstb[q],
                                      sid[q]).wait()
                pltpu.async_copy(xl_hbm.at[srcb[q]], xlb[q], sgx[q])
                pltpu.async_copy(xr_hbm.at[dstb[q]], xrb[q], sgr[q])
            if p == 0:
                _prefetch_idx()
            else:
                @pl.when(t2 < (_CB // 2) - 1)
                def _():
                    _prefetch_idx()

            pltpu.make_async_copy(xl_hbm.at[srcb[p]], xlb[p], sgx[p]).wait()
            pltpu.make_async_copy(xr_hbm.at[dstb[p]], xrb[p], sgr[p]).wait()
            if p == 0:
                _fire_gathers()
            else:
                @pl.when(t2 < (_CB // 2) - 1)
                def _():
                    _fire_gathers()
            _compute(p)
            pltpu.sync_copy(ab[p], spden.at[dstb[p]], add=True)
            pltpu.sync_copy(xlb[p], spout.at[dstb[p]], add=True)
        return carry
    lax.fori_loop(0, _CB // 2, _pair, 0)

    plsc.subcore_barrier()

    @pl.when((c == 0) & (s == 0))
    def _():
        pltpu.sync_copy(spden, den0_hbm)

    @pl.when((c == 1) & (s == 0))
    def _():
        pltpu.sync_copy(spden, den1_hbm)

    for off, sz in _CHUNKS:
        r0 = base + off
        pltpu.sync_copy(spout.at[pl.ds(r0, sz)],
                        raw_hbm.at[c, pl.ds(r0, sz)])


def _sc_edge_pass(xl, xr, src3, dst3, att):
    attb = att
    mesh = plsc.VectorSubcoreMesh(core_axis_name="c", subcore_axis_name="s")
    kern = pl.kernel(
        _sc_body,
        mesh=mesh,
        compiler_params=pltpu.CompilerParams(needs_layout_passes=False),
        out_type=[
            jax.ShapeDtypeStruct((2, _NP, _D), jnp.float32),
            jax.ShapeDtypeStruct((_NP,), jnp.float32),
            jax.ShapeDtypeStruct((_NP,), jnp.float32),
        ],
        scratch_types=(
            [pltpu.VMEM((_LB,), jnp.int32)] * 4
            + [pltpu.VMEM((_LB, _D), jnp.float32)] * 4
            + [pltpu.VMEM((_LB,), jnp.float32)] * 2
            + [pltpu.VMEM((_D,), jnp.float32)]
            + [pltpu.VMEM_SHARED((_NP, _D), jnp.float32),
               pltpu.VMEM_SHARED((_NP,), jnp.float32)]
            + [pltpu.SemaphoreType.DMA] * 10
        ),
    )
    raw, den0, den1 = kern(xl, xr, src3, dst3, attb)
    return raw, jnp.stack([den0, den1])


def kernel(x, edge_index, Wl1, bl1, Wr1, br1, att1, bias1,
           Wl2, bl2, Wr2, br2, att2, bias2):
    n = _N
    loop = jnp.arange(n, dtype=edge_index.dtype)
    pad = _EP - (edge_index.shape[1] + n)
    padv = (n + jnp.arange(pad, dtype=edge_index.dtype) % (_NP - n))
    src = jnp.concatenate([edge_index[0], loop, padv])
    dst = jnp.concatenate([edge_index[1], loop, padv])
    src3 = src.reshape(_NW, _CB, _LB)
    dst3 = dst.reshape(_NW, _CB, _LB)

    x_pad = jnp.pad(x, ((0, _NP - n), (0, 0)))

    xl1, xr1 = _tc_transform(x_pad, Wl1, bl1, Wr1, br1)
    raw1, den1 = _sc_edge_pass(xl1, xr1, src3, dst3, att1)
    xl2, xr2 = _tc_combine_transform(raw1, den1, bias1, Wl2, bl2, Wr2, br2)
    raw2, den2 = _sc_edge_pass(xl2, xr2, src3, dst3, att2)
    out = _tc_final(raw2, den2, bias2)
    return out[:n]
